# baseline (speedup 1.0000x reference)
import functools

import jax
import jax.numpy as jnp
from jax import lax
from jax.experimental import pallas as pl
from jax.experimental.pallas import tpu as pltpu
from jax.experimental.pallas import tpu_sc as plsc

HIDDEN = 2048
NUM_EXPERTS = 16
TOP_K = 2
NUM_CORES = 2
NUM_SUBCORES = 16
LANES = 16

TOKENS = 16384
NUM_WORKERS = NUM_CORES * NUM_SUBCORES
NCHUNK = 1
CTOK = TOKENS // NCHUNK
TOK_PER_W = CTOK // NUM_WORKERS
GROUPS = TOK_PER_W // LANES


def _logits_body(x_ref, w_ref, o_ref):
    o_ref[...] = lax.dot_general(
        x_ref[...], w_ref[...],
        dimension_numbers=(((1,), (1,)), ((), ())),
        preferred_element_type=jnp.float32,
    )


def _fill_body(l_ref, z_ref):
    z_ref[...] = jnp.zeros_like(z_ref)


def _zeros_fill(logits):
    bz = 2048
    return pl.pallas_call(
        _fill_body,
        grid=(TOKENS // bz,),
        in_specs=[pl.BlockSpec((bz, NUM_EXPERTS), lambda i: (i, 0))],
        out_specs=pl.BlockSpec((bz, HIDDEN), lambda i: (i, 0)),
        out_shape=jax.ShapeDtypeStruct((TOKENS, HIDDEN), jnp.float32),
    )(logits)


def _sc_mask_body(logits_hbm, mask_hbm, lv, m):
    c = lax.axis_index("c")
    s = lax.axis_index("s")
    wid = s * NUM_CORES + c
    base = wid * TOK_PER_W
    pltpu.sync_copy(logits_hbm.at[pl.ds(base, TOK_PER_W), :], lv)

    lanes = lax.broadcasted_iota(jnp.int32, (LANES,), 0)
    neg_inf = jnp.full((LANES,), -jnp.inf, jnp.float32)
    zero_i = jnp.zeros((LANES,), jnp.int32)
    one_f = jnp.ones((LANES,), jnp.float32)
    zero_f = jnp.zeros((LANES,), jnp.float32)

    def g_body(g, carry):
        row = g * LANES + lanes
        m1, e1 = neg_inf, zero_i
        m2, e2 = neg_inf, zero_i
        for e in range(NUM_EXPERTS):
            col = plsc.load_gather(lv, [row, jnp.full((LANES,), e, jnp.int32)])
            ev = jnp.full((LANES,), e, jnp.int32)
            gt1 = col > m1
            gt2 = col > m2
            m2 = jnp.where(gt1, m1, jnp.where(gt2, col, m2))
            e2 = jnp.where(gt1, e1, jnp.where(gt2, ev, e2))
            m1 = jnp.where(gt1, col, m1)
            e1 = jnp.where(gt1, ev, e1)
        for e in range(NUM_EXPERTS):
            m[e, 0, pl.ds(g * LANES, LANES)] = jnp.where(e1 == e, one_f, zero_f)
            m[e, 1, pl.ds(g * LANES, LANES)] = jnp.where(e2 == e, one_f, zero_f)
        return carry

    lax.fori_loop(0, GROUPS, g_body, 0)
    pltpu.sync_copy(m, mask_hbm.at[:, :, pl.ds(base, TOK_PER_W)])


def _expert_mask(logits):
    mesh = plsc.VectorSubcoreMesh(
        core_axis_name="c", subcore_axis_name="s",
        num_cores=NUM_CORES, num_subcores=NUM_SUBCORES,
    )
    f = pl.kernel(
        _sc_mask_body,
        out_type=jax.ShapeDtypeStruct((NUM_EXPERTS, TOP_K, CTOK), jnp.float32),
        mesh=mesh,
        scratch_types=[
            pltpu.VMEM((TOK_PER_W, NUM_EXPERTS), jnp.float32),
            pltpu.VMEM((NUM_EXPERTS, TOP_K, TOK_PER_W), jnp.float32),
        ],
        compiler_params=pltpu.CompilerParams(needs_layout_passes=False),
    )
    return f(logits)


def kernel(hidden_states, gate_w):
    x = hidden_states.reshape(-1, HIDDEN)
    bt = 2048
    mask_chunks = []
    logits_list = []
    for ci in range(NCHUNK):
        off = ci * (CTOK // bt)
        logits = pl.pallas_call(
            _logits_body,
            grid=(CTOK // bt,),
            in_specs=[
                pl.BlockSpec((bt, HIDDEN), lambda i, o=off: (i + o, 0)),
                pl.BlockSpec((NUM_EXPERTS, HIDDEN), lambda i: (0, 0)),
            ],
            out_specs=pl.BlockSpec((bt, NUM_EXPERTS), lambda i: (i, 0)),
            out_shape=jax.ShapeDtypeStruct((CTOK, NUM_EXPERTS), jnp.float32),
        )(x, gate_w)
        logits_list.append(logits)
        mask_chunks.append(_expert_mask(logits))
    expert_mask = (mask_chunks[0] if NCHUNK == 1
                   else jnp.concatenate(mask_chunks, axis=2))
    final_hidden_states = _zeros_fill(logits_list[-1])
    return (final_hidden_states, expert_mask)


# speedup vs baseline: 1.0002x; 1.0002x over previous
import functools

import jax
import jax.numpy as jnp
from jax import lax
from jax.experimental import pallas as pl
from jax.experimental.pallas import tpu as pltpu
from jax.experimental.pallas import tpu_sc as plsc

HIDDEN = 2048
NUM_EXPERTS = 16
TOP_K = 2
NUM_CORES = 2
NUM_SUBCORES = 16
LANES = 16

TOKENS = 16384
NUM_WORKERS = NUM_CORES * NUM_SUBCORES
NCHUNK = 1
CTOK = TOKENS // NCHUNK
TOK_PER_W = CTOK // NUM_WORKERS
GROUPS = TOK_PER_W // LANES


def _logits_body(x_ref, w_ref, o_ref):
    o_ref[...] = lax.dot_general(
        x_ref[...], w_ref[...],
        dimension_numbers=(((1,), (1,)), ((), ())),
        preferred_element_type=jnp.float32,
    )


def _fill_body(l_ref, z_ref):
    z_ref[...] = jnp.zeros_like(z_ref)


def _zeros_fill(logits):
    bz = 1024
    return pl.pallas_call(
        _fill_body,
        grid=(TOKENS // bz,),
        in_specs=[pl.BlockSpec((bz, NUM_EXPERTS), lambda i: (i, 0))],
        out_specs=pl.BlockSpec((bz, HIDDEN), lambda i: (i, 0)),
        out_shape=jax.ShapeDtypeStruct((TOKENS, HIDDEN), jnp.float32),
    )(logits)


def _sc_mask_body(logits_hbm, mask_hbm, lv, m):
    c = lax.axis_index("c")
    s = lax.axis_index("s")
    wid = s * NUM_CORES + c
    base = wid * TOK_PER_W
    pltpu.sync_copy(logits_hbm.at[pl.ds(base, TOK_PER_W), :], lv)

    lanes = lax.broadcasted_iota(jnp.int32, (LANES,), 0)
    neg_inf = jnp.full((LANES,), -jnp.inf, jnp.float32)
    zero_i = jnp.zeros((LANES,), jnp.int32)
    one_f = jnp.ones((LANES,), jnp.float32)
    zero_f = jnp.zeros((LANES,), jnp.float32)

    def g_body(g, carry):
        row = g * LANES + lanes
        m1, e1 = neg_inf, zero_i
        m2, e2 = neg_inf, zero_i
        for e in range(NUM_EXPERTS):
            col = plsc.load_gather(lv, [row, jnp.full((LANES,), e, jnp.int32)])
            ev = jnp.full((LANES,), e, jnp.int32)
            gt1 = col > m1
            gt2 = col > m2
            m2 = jnp.where(gt1, m1, jnp.where(gt2, col, m2))
            e2 = jnp.where(gt1, e1, jnp.where(gt2, ev, e2))
            m1 = jnp.where(gt1, col, m1)
            e1 = jnp.where(gt1, ev, e1)
        for e in range(NUM_EXPERTS):
            m[e, 0, pl.ds(g * LANES, LANES)] = jnp.where(e1 == e, one_f, zero_f)
            m[e, 1, pl.ds(g * LANES, LANES)] = jnp.where(e2 == e, one_f, zero_f)
        return carry

    lax.fori_loop(0, GROUPS, g_body, 0)
    pltpu.sync_copy(m, mask_hbm.at[:, :, pl.ds(base, TOK_PER_W)])


def _expert_mask(logits):
    mesh = plsc.VectorSubcoreMesh(
        core_axis_name="c", subcore_axis_name="s",
        num_cores=NUM_CORES, num_subcores=NUM_SUBCORES,
    )
    f = pl.kernel(
        _sc_mask_body,
        out_type=jax.ShapeDtypeStruct((NUM_EXPERTS, TOP_K, CTOK), jnp.float32),
        mesh=mesh,
        scratch_types=[
            pltpu.VMEM((TOK_PER_W, NUM_EXPERTS), jnp.float32),
            pltpu.VMEM((NUM_EXPERTS, TOP_K, TOK_PER_W), jnp.float32),
        ],
        compiler_params=pltpu.CompilerParams(needs_layout_passes=False),
    )
    return f(logits)


def kernel(hidden_states, gate_w):
    x = hidden_states.reshape(-1, HIDDEN)
    bt = 1024
    mask_chunks = []
    logits_list = []
    for ci in range(NCHUNK):
        off = ci * (CTOK // bt)
        logits = pl.pallas_call(
            _logits_body,
            grid=(CTOK // bt,),
            in_specs=[
                pl.BlockSpec((bt, HIDDEN), lambda i, o=off: (i + o, 0)),
                pl.BlockSpec((NUM_EXPERTS, HIDDEN), lambda i: (0, 0)),
            ],
            out_specs=pl.BlockSpec((bt, NUM_EXPERTS), lambda i: (i, 0)),
            out_shape=jax.ShapeDtypeStruct((CTOK, NUM_EXPERTS), jnp.float32),
        )(x, gate_w)
        logits_list.append(logits)
        mask_chunks.append(_expert_mask(logits))
    expert_mask = (mask_chunks[0] if NCHUNK == 1
                   else jnp.concatenate(mask_chunks, axis=2))
    final_hidden_states = _zeros_fill(logits_list[-1])
    return (final_hidden_states, expert_mask)


# speedup vs baseline: 1.0359x; 1.0357x over previous
import functools

import jax
import jax.numpy as jnp
from jax import lax
from jax.experimental import pallas as pl
from jax.experimental.pallas import tpu as pltpu
from jax.experimental.pallas import tpu_sc as plsc

HIDDEN = 2048
NUM_EXPERTS = 16
TOP_K = 2
NUM_CORES = 2
NUM_SUBCORES = 16
LANES = 16

TOKENS = 16384
NUM_WORKERS = NUM_CORES * NUM_SUBCORES
NCHUNK = 1
CTOK = TOKENS // NCHUNK
TOK_PER_W = CTOK // NUM_WORKERS
GROUPS = TOK_PER_W // LANES


def _logits_body(x_ref, w_ref, o_ref):
    o_ref[...] = lax.dot_general(
        x_ref[...], w_ref[...],
        dimension_numbers=(((1,), (1,)), ((), ())),
        preferred_element_type=jnp.float32,
    )


def _fill_body(l_ref, z_ref):
    z_ref[...] = jnp.zeros_like(z_ref)


def _zeros_fill(logits):
    bz = 2048
    return pl.pallas_call(
        _fill_body,
        grid=(TOKENS // bz,),
        in_specs=[pl.BlockSpec((8, NUM_EXPERTS), lambda i: (0, 0))],
        out_specs=pl.BlockSpec((bz, HIDDEN), lambda i: (i, 0)),
        out_shape=jax.ShapeDtypeStruct((TOKENS, HIDDEN), jnp.float32),
    )(logits)


def _sc_mask_body(logits_hbm, mask_hbm, lv, m):
    c = lax.axis_index("c")
    s = lax.axis_index("s")
    wid = s * NUM_CORES + c
    base = wid * TOK_PER_W
    pltpu.sync_copy(logits_hbm.at[pl.ds(base, TOK_PER_W), :], lv)

    lanes = lax.broadcasted_iota(jnp.int32, (LANES,), 0)
    neg_inf = jnp.full((LANES,), -jnp.inf, jnp.float32)
    zero_i = jnp.zeros((LANES,), jnp.int32)
    one_f = jnp.ones((LANES,), jnp.float32)
    zero_f = jnp.zeros((LANES,), jnp.float32)

    def g_body(g, carry):
        row = g * LANES + lanes
        m1, e1 = neg_inf, zero_i
        m2, e2 = neg_inf, zero_i
        for e in range(NUM_EXPERTS):
            col = plsc.load_gather(lv, [row, jnp.full((LANES,), e, jnp.int32)])
            ev = jnp.full((LANES,), e, jnp.int32)
            gt1 = col > m1
            gt2 = col > m2
            m2 = jnp.where(gt1, m1, jnp.where(gt2, col, m2))
            e2 = jnp.where(gt1, e1, jnp.where(gt2, ev, e2))
            m1 = jnp.where(gt1, col, m1)
            e1 = jnp.where(gt1, ev, e1)
        for e in range(NUM_EXPERTS):
            m[e, 0, pl.ds(g * LANES, LANES)] = jnp.where(e1 == e, one_f, zero_f)
            m[e, 1, pl.ds(g * LANES, LANES)] = jnp.where(e2 == e, one_f, zero_f)
        return carry

    lax.fori_loop(0, GROUPS, g_body, 0)
    pltpu.sync_copy(m, mask_hbm.at[:, :, pl.ds(base, TOK_PER_W)])


def _expert_mask(logits):
    mesh = plsc.VectorSubcoreMesh(
        core_axis_name="c", subcore_axis_name="s",
        num_cores=NUM_CORES, num_subcores=NUM_SUBCORES,
    )
    f = pl.kernel(
        _sc_mask_body,
        out_type=jax.ShapeDtypeStruct((NUM_EXPERTS, TOP_K, CTOK), jnp.float32),
        mesh=mesh,
        scratch_types=[
            pltpu.VMEM((TOK_PER_W, NUM_EXPERTS), jnp.float32),
            pltpu.VMEM((NUM_EXPERTS, TOP_K, TOK_PER_W), jnp.float32),
        ],
        compiler_params=pltpu.CompilerParams(needs_layout_passes=False, skip_device_barrier=True),
    )
    return f(logits)


def kernel(hidden_states, gate_w):
    x = hidden_states.reshape(-1, HIDDEN)
    bt = 1024
    mask_chunks = []
    logits_list = []
    for ci in range(NCHUNK):
        off = ci * (CTOK // bt)
        logits = pl.pallas_call(
            _logits_body,
            grid=(CTOK // bt,),
            in_specs=[
                pl.BlockSpec((bt, HIDDEN), lambda i, o=off: (i + o, 0)),
                pl.BlockSpec((NUM_EXPERTS, HIDDEN), lambda i: (0, 0)),
            ],
            out_specs=pl.BlockSpec((bt, NUM_EXPERTS), lambda i: (i, 0)),
            out_shape=jax.ShapeDtypeStruct((CTOK, NUM_EXPERTS), jnp.float32),
        )(x, gate_w)
        logits_list.append(logits)
        mask_chunks.append(_expert_mask(logits))
    expert_mask = (mask_chunks[0] if NCHUNK == 1
                   else jnp.concatenate(mask_chunks, axis=2))
    final_hidden_states = _zeros_fill(logits_list[-1])
    return (final_hidden_states, expert_mask)


# speedup vs baseline: 1.0387x; 1.0027x over previous
import jax
import jax.numpy as jnp
from jax import lax
from jax.experimental import pallas as pl
from jax.experimental.pallas import tpu as pltpu
from jax.experimental.pallas import tpu_sc as plsc

HIDDEN = 2048
NUM_EXPERTS = 16
TOP_K = 2
NUM_CORES = 2
NUM_SUBCORES = 16
LANES = 16

TOKENS = 16384
NUM_WORKERS = NUM_CORES * NUM_SUBCORES
NCHUNK = 1
CTOK = TOKENS // NCHUNK
TOK_PER_W = CTOK // NUM_WORKERS
GROUPS = TOK_PER_W // LANES


def _logits_body(x_ref, w_ref, o_ref):
    o_ref[...] = lax.dot_general(
        x_ref[...], w_ref[...],
        dimension_numbers=(((1,), (1,)), ((), ())),
        preferred_element_type=jnp.float32,
    )


def _fill_body(l_ref, z_ref):
    z_ref[...] = jnp.zeros_like(z_ref)


def _zeros_fill(logits):
    bz = 2048
    return pl.pallas_call(
        _fill_body,
        grid=(TOKENS // bz,),
        in_specs=[pl.BlockSpec((8, NUM_EXPERTS), lambda i: (0, 0))],
        out_specs=pl.BlockSpec((bz, HIDDEN), lambda i: (i, 0)),
        out_shape=jax.ShapeDtypeStruct((TOKENS, HIDDEN), jnp.float32),
    )(logits)


def _sc_mask_body(logits_hbm, mask_hbm, lv, m):
    c = lax.axis_index("c")
    s = lax.axis_index("s")
    wid = s * NUM_CORES + c
    base = wid * TOK_PER_W
    pltpu.sync_copy(logits_hbm.at[pl.ds(base, TOK_PER_W), :], lv)

    lanes = lax.broadcasted_iota(jnp.int32, (LANES,), 0)
    neg_inf = jnp.full((LANES,), -jnp.inf, jnp.float32)
    zero_i = jnp.zeros((LANES,), jnp.int32)
    one_f = jnp.ones((LANES,), jnp.float32)
    zero_f = jnp.zeros((LANES,), jnp.float32)

    def g_body(g, carry):
        row = g * LANES + lanes
        m1, e1 = neg_inf, zero_i
        m2, e2 = neg_inf, zero_i
        for e in range(NUM_EXPERTS):
            col = plsc.load_gather(lv, [row, jnp.full((LANES,), e, jnp.int32)])
            ev = jnp.full((LANES,), e, jnp.int32)
            gt1 = col > m1
            gt2 = col > m2
            m2 = jnp.where(gt1, m1, jnp.where(gt2, col, m2))
            e2 = jnp.where(gt1, e1, jnp.where(gt2, ev, e2))
            m1 = jnp.where(gt1, col, m1)
            e1 = jnp.where(gt1, ev, e1)
        for e in range(NUM_EXPERTS):
            m[e, 0, pl.ds(g * LANES, LANES)] = jnp.where(e1 == e, one_f, zero_f)
            m[e, 1, pl.ds(g * LANES, LANES)] = jnp.where(e2 == e, one_f, zero_f)
        return carry

    lax.fori_loop(0, GROUPS, g_body, 0)
    pltpu.sync_copy(m, mask_hbm.at[:, :, pl.ds(base, TOK_PER_W)])


def _expert_mask(logits):
    mesh = plsc.VectorSubcoreMesh(
        core_axis_name="c", subcore_axis_name="s",
        num_cores=NUM_CORES, num_subcores=NUM_SUBCORES,
    )
    f = pl.kernel(
        _sc_mask_body,
        out_type=jax.ShapeDtypeStruct((NUM_EXPERTS, TOP_K, CTOK), jnp.float32),
        mesh=mesh,
        scratch_types=[
            pltpu.VMEM((TOK_PER_W, NUM_EXPERTS), jnp.float32),
            pltpu.VMEM((NUM_EXPERTS, TOP_K, TOK_PER_W), jnp.float32),
        ],
        compiler_params=pltpu.CompilerParams(needs_layout_passes=False),
    )
    return f(logits)


def kernel(hidden_states, gate_w):
    x = hidden_states.reshape(-1, HIDDEN)
    bt = 1024
    mask_chunks = []
    logits_list = []
    for ci in range(NCHUNK):
        off = ci * (CTOK // bt)
        logits = pl.pallas_call(
            _logits_body,
            grid=(CTOK // bt,),
            in_specs=[
                pl.BlockSpec((bt, HIDDEN), lambda i, o=off: (i + o, 0)),
                pl.BlockSpec((NUM_EXPERTS, HIDDEN), lambda i: (0, 0)),
            ],
            out_specs=pl.BlockSpec((bt, NUM_EXPERTS), lambda i: (i, 0)),
            out_shape=jax.ShapeDtypeStruct((CTOK, NUM_EXPERTS), jnp.float32),
        )(x, gate_w)
        logits_list.append(logits)
        mask_chunks.append(_expert_mask(logits))
    expert_mask = (mask_chunks[0] if NCHUNK == 1
                   else jnp.concatenate(mask_chunks, axis=2))
    final_hidden_states = _zeros_fill(logits_list[-1])
    return (final_hidden_states, expert_mask)
